# baseline (speedup 1.0000x reference)
import jax
import jax.numpy as jnp
from jax import lax
from jax.experimental import pallas as pl
from jax.experimental.pallas import tpu as pltpu
from jax.experimental.pallas import tpu_sc as plsc

D_MODEL = 128
WINDOW = 128
NBUF = 5
LAG = 3
N_UNITS = 32


def kernel(x, table):
    b, s = x.shape
    n = b * s
    per_unit = n // N_UNITS
    nsteps = per_unit // WINDOW
    rounds = nsteps // NBUF
    idx = x.reshape(n).astype(jnp.int32)

    mesh = plsc.VectorSubcoreMesh(core_axis_name="core",
                                  subcore_axis_name="subcore")

    @pl.kernel(out_type=jax.ShapeDtypeStruct((n, D_MODEL), table.dtype),
               mesh=mesh,
               scratch_types=[pltpu.VMEM((per_unit,), jnp.int32),
                              pltpu.VMEM((NBUF, WINDOW, D_MODEL),
                                         jnp.float32),
                              pltpu.SemaphoreType.DMA((NBUF,)),
                              pltpu.SemaphoreType.DMA((NBUF,)),
                              pltpu.SemaphoreType.DMA])
    def gather_kernel(table_hbm, idx_hbm, out_hbm, idx_v, rows_v, gsem,
                      wsem, isem):
        wid = lax.axis_index("subcore") * 2 + lax.axis_index("core")
        unit_base = wid * per_unit

        pltpu.async_copy(idx_hbm.at[pl.ds(unit_base, per_unit)], idx_v,
                         isem).wait()

        def gather(slot, step):
            return pltpu.make_async_copy(
                table_hbm.at[idx_v.at[pl.ds(step * WINDOW, WINDOW)]],
                rows_v.at[slot], gsem.at[slot])

        def wb(slot, step):
            return pltpu.make_async_copy(
                rows_v.at[slot],
                out_hbm.at[pl.ds(unit_base + step * WINDOW, WINDOW)],
                wsem.at[slot])

        for t in range(LAG):
            gather(t % NBUF, t).start()
        for t in range(LAG, NBUF):
            gather(t % NBUF, t).start()
            gather((t - LAG) % NBUF, t - LAG).wait()
            wb((t - LAG) % NBUF, t - LAG).start()

        @pl.loop(1, rounds)
        def _(r):
            for slot in range(NBUF):
                t = r * NBUF + slot
                wb(slot, t - NBUF).wait()
                gather(slot, t).start()
                s2 = (slot + NBUF - LAG) % NBUF
                gather(s2, t - LAG).wait()
                wb(s2, t - LAG).start()

        for step in range(nsteps, nsteps + LAG):
            gather((step - LAG) % NBUF, step - LAG).wait()
            wb((step - LAG) % NBUF, step - LAG).start()
        for step in range(nsteps - NBUF, nsteps):
            wb(step % NBUF, step).wait()

    out = gather_kernel(table, idx)
    return out.reshape(b, s, D_MODEL)


# speedup vs baseline: 1.0021x; 1.0021x over previous
import jax
import jax.numpy as jnp
from jax import lax
from jax.experimental import pallas as pl
from jax.experimental.pallas import tpu as pltpu
from jax.experimental.pallas import tpu_sc as plsc

D_MODEL = 128
WINDOW = 128
NBUF = 6
LAG = 4
N_UNITS = 32


def kernel(x, table):
    b, s = x.shape
    n = b * s
    per_unit = n // N_UNITS
    nsteps = per_unit // WINDOW
    rounds = nsteps // NBUF
    idx = x.reshape(n).astype(jnp.int32)

    mesh = plsc.VectorSubcoreMesh(core_axis_name="core",
                                  subcore_axis_name="subcore")

    @pl.kernel(out_type=jax.ShapeDtypeStruct((n, D_MODEL), table.dtype),
               mesh=mesh,
               scratch_types=[pltpu.VMEM((per_unit,), jnp.int32),
                              pltpu.VMEM((NBUF, WINDOW, D_MODEL),
                                         jnp.float32),
                              pltpu.SemaphoreType.DMA((NBUF,)),
                              pltpu.SemaphoreType.DMA((NBUF,)),
                              pltpu.SemaphoreType.DMA])
    def gather_kernel(table_hbm, idx_hbm, out_hbm, idx_v, rows_v, gsem,
                      wsem, isem):
        wid = lax.axis_index("subcore") * 2 + lax.axis_index("core")
        unit_base = wid * per_unit

        pltpu.async_copy(idx_hbm.at[pl.ds(unit_base, per_unit)], idx_v,
                         isem).wait()

        def gather(slot, step):
            return pltpu.make_async_copy(
                table_hbm.at[idx_v.at[pl.ds(step * WINDOW, WINDOW)]],
                rows_v.at[slot], gsem.at[slot])

        def wb(slot, step):
            return pltpu.make_async_copy(
                rows_v.at[slot],
                out_hbm.at[pl.ds(unit_base + step * WINDOW, WINDOW)],
                wsem.at[slot])

        for t in range(LAG):
            gather(t % NBUF, t).start()
        for t in range(LAG, NBUF):
            gather(t % NBUF, t).start()
            gather((t - LAG) % NBUF, t - LAG).wait()
            wb((t - LAG) % NBUF, t - LAG).start()

        @pl.loop(1, rounds)
        def _(r):
            for slot in range(NBUF):
                t = r * NBUF + slot
                wb(slot, t - NBUF).wait()
                gather(slot, t).start()
                s2 = (slot + NBUF - LAG) % NBUF
                gather(s2, t - LAG).wait()
                wb(s2, t - LAG).start()

        for t in range(NBUF * rounds, nsteps):
            slot = t % NBUF
            wb(slot, t - NBUF).wait()
            gather(slot, t).start()
            s2 = (t - LAG) % NBUF
            gather(s2, t - LAG).wait()
            wb(s2, t - LAG).start()

        for step in range(nsteps, nsteps + LAG):
            gather((step - LAG) % NBUF, step - LAG).wait()
            wb((step - LAG) % NBUF, step - LAG).start()
        for step in range(nsteps - NBUF, nsteps):
            wb(step % NBUF, step).wait()

    out = gather_kernel(table, idx)
    return out.reshape(b, s, D_MODEL)
